# baseline (speedup 1.0000x reference)
import functools

import jax
import jax.numpy as jnp
from jax import lax
from jax.experimental import pallas as pl
from jax.experimental.pallas import tpu as pltpu
from jax.experimental.pallas import tpu_sc as plsc

R = 0.2
MAX_K = 20
H1, C1 = 9, 27
H2, C2 = 9, 64
BN_EPS = 1e-5
NUM_CLASSES = 40
N_REAL = 10000
NP_PAD = 10240
BLK = 128
KSLOT = MAX_K + 1
HI = jax.lax.Precision.HIGHEST


def _topk_body(xT_ref, nsqT_ref, batT_ref, xb_ref, batb_ref,
               nbr_ref, rowmx_ref, cnt_ref, score_ref, *, r2, n_real):
    blk = xb_ref.shape[0]
    npad = xT_ref.shape[1]
    i0 = pl.program_id(0)
    xb = xb_ref[...]
    mm = jax.lax.dot_general(
        xb, xT_ref[...], (((1,), (0,)), ((), ())),
        precision=HI, preferred_element_type=jnp.float32)
    nb = jnp.sum(xb * xb, axis=1, keepdims=True)
    d2 = jnp.maximum(nb + nsqT_ref[...] - 2.0 * mm, 0.0)
    valid = (d2 <= r2) & (batb_ref[...] == batT_ref[...])
    cnt_ref[...] = jnp.sum(valid.astype(jnp.int32), axis=1, keepdims=True)
    score_ref[...] = jnp.where(valid, -d2, -jnp.inf)

    colid = jax.lax.broadcasted_iota(jnp.int32, (blk, npad), 1)
    qvec = i0 * blk + jax.lax.broadcasted_iota(jnp.int32, (blk, 1), 0)

    rowmx = jnp.full((blk, 1), -1, jnp.int32)
    for k in range(MAX_K):
        s = score_ref[...]
        m = jnp.max(s, axis=1, keepdims=True)
        found = m > -jnp.inf
        cand = jnp.where(s == m, colid, npad)
        idx = jnp.min(cand, axis=1, keepdims=True)
        score_ref[...] = jnp.where(colid == idx, -jnp.inf, s)
        keep = found & (idx != qvec)
        nbr_ref[:, k:k + 1] = jnp.where(keep, idx, n_real)
        rowmx = jnp.maximum(rowmx, jnp.where(keep, jnp.maximum(idx, qvec), -1))
    nbr_ref[:, MAX_K:] = jnp.full((blk, 32 - MAX_K), n_real, jnp.int32)
    rowmx_ref[...] = rowmx


def _radius_topk_call(xp, batp):
    N, d = N_REAL, xp.shape[1]
    xT = xp.T
    nsqT = jnp.sum(xT * xT, axis=0, keepdims=True)
    batT = batp.reshape(1, NP_PAD)
    batb = batp.reshape(NP_PAD, 1)
    grid = NP_PAD // BLK
    body = functools.partial(_topk_body, r2=R * R, n_real=N)
    nbr, rowmx, cnt = pl.pallas_call(
        body,
        grid=(grid,),
        in_specs=[
            pl.BlockSpec((d, NP_PAD), lambda i: (0, 0)),
            pl.BlockSpec((1, NP_PAD), lambda i: (0, 0)),
            pl.BlockSpec((1, NP_PAD), lambda i: (0, 0)),
            pl.BlockSpec((BLK, d), lambda i: (i, 0)),
            pl.BlockSpec((BLK, 1), lambda i: (i, 0)),
        ],
        out_specs=[
            pl.BlockSpec((BLK, 32), lambda i: (i, 0)),
            pl.BlockSpec((BLK, 1), lambda i: (i, 0)),
            pl.BlockSpec((BLK, 1), lambda i: (i, 0)),
        ],
        out_shape=[
            jax.ShapeDtypeStruct((NP_PAD, 32), jnp.int32),
            jax.ShapeDtypeStruct((NP_PAD, 1), jnp.int32),
            jax.ShapeDtypeStruct((NP_PAD, 1), jnp.int32),
        ],
        scratch_shapes=[pltpu.VMEM((BLK, NP_PAD), jnp.float32)],
    )(xT, nsqT, batT, xp, batb)
    return nbr, rowmx, cnt


def _pad_inputs(x, batch):
    N, d = x.shape
    xp = jnp.zeros((NP_PAD, d), jnp.float32).at[:N].set(x)
    batp = jnp.full((NP_PAD,), -1, jnp.int32) - jnp.arange(NP_PAD, dtype=jnp.int32)
    batp = batp.at[:N].set(batch.astype(jnp.int32))
    return xp, batp


def _xw_body(x_ref, w_ref, o_ref):
    o_ref[...] = jax.lax.dot_general(
        x_ref[...], w_ref[...], (((1,), (0,)), ((), ())),
        precision=HI, preferred_element_type=jnp.float32)


def _xw_call(xp, Wp):
    d, Dp = Wp.shape
    grid = NP_PAD // 512
    return pl.pallas_call(
        _xw_body,
        grid=(grid,),
        in_specs=[pl.BlockSpec((512, d), lambda i: (i, 0)),
                  pl.BlockSpec((d, Dp), lambda i: (0, 0))],
        out_specs=pl.BlockSpec((512, Dp), lambda i: (i, 0)),
        out_shape=jax.ShapeDtypeStruct((NP_PAD, Dp), jnp.float32),
    )(xp, Wp)


def _sc_gather(table, idx):
    B = idx.shape[0]
    Dp = table.shape[1]
    NW = 32
    b_per_w = B // NW
    chunk = 120
    while b_per_w % chunk:
        chunk -= 8
    nch = b_per_w // chunk
    idx3 = idx.reshape(NW, nch, chunk)
    mesh = plsc.VectorSubcoreMesh(core_axis_name="c", subcore_axis_name="s")

    @functools.partial(
        pl.kernel, mesh=mesh,
        out_type=jax.ShapeDtypeStruct((B, Dp), jnp.float32),
        scratch_types=[
            pltpu.VMEM((chunk,), jnp.int32),
            pltpu.VMEM((chunk, Dp), jnp.float32),
            pltpu.SemaphoreType.DMA,
        ],
    )
    def k(table_hbm, idx_hbm, out_hbm, idx_v, rows_v, sem):
        wid = lax.axis_index("s") * 2 + lax.axis_index("c")
        base = wid * b_per_w

        def body(j, carry):
            pltpu.sync_copy(idx_hbm.at[wid, j], idx_v)
            pltpu.async_copy(table_hbm.at[idx_v], rows_v, sem).wait()
            off = pl.multiple_of(base + j * chunk, 8)
            pltpu.sync_copy(rows_v, out_hbm.at[pl.ds(off, chunk)])
            return carry

        lax.fori_loop(0, nch, body, 0)

    return k(table, idx3)


def _att_body(xj_ref, xw_ref, nbr_ref, nn_ref, lw_ref, lb_ref, o_ref, agg_ref,
              *, heads, cp, c_true):
    blk = nbr_ref.shape[0]
    i0 = pl.program_id(0)
    nn = nn_ref[0]
    rowid = i0 * blk + jax.lax.broadcasted_iota(jnp.int32, (blk, 1), 0)
    maskk = [nbr_ref[:, k:k + 1] != N_REAL for k in range(MAX_K)]
    maskk.append(rowid < nn)

    xjk = [xj_ref[k] for k in range(KSLOT)]
    yk = []
    for k in range(KSLOT):
        acc = xjk[k][:, 0 * cp:1 * cp]
        for h in range(1, heads):
            acc = acc + xjk[k][:, h * cp:(h + 1) * cp]
        yk.append(acc)
    xw = xw_ref[...]

    inv_sqrt = 1.0 / jnp.sqrt(jnp.asarray(c_true, jnp.float32))
    neg = jnp.float32(-jnp.inf)
    for h in range(heads):
        xwh = xw[:, h * cp:(h + 1) * cp]
        sck = [jnp.where(maskk[k],
                         jnp.sum(yk[k] * xwh, axis=1, keepdims=True) * inv_sqrt,
                         neg) for k in range(KSLOT)]
        m = sck[0]
        for k in range(1, KSLOT):
            m = jnp.maximum(m, sck[k])
        m = jnp.where(jnp.isfinite(m), m, 0.0)
        ek = [jnp.where(maskk[k], jnp.exp(sck[k] - m), 0.0) for k in range(KSLOT)]
        s = ek[0]
        for k in range(1, KSLOT):
            s = s + ek[k]
        sinv = 1.0 / (s + 1e-16)
        agg_h = jnp.full((blk, cp), neg, jnp.float32)
        for k in range(KSLOT):
            msg_k = jnp.where(maskk[k],
                              (ek[k] * sinv) * xjk[k][:, h * cp:(h + 1) * cp],
                              neg)
            agg_h = jnp.maximum(agg_h, msg_k)
        agg_ref[:, h * cp:(h + 1) * cp] = jnp.where(
            jnp.isfinite(agg_h), agg_h, 0.0)
    o_ref[...] = jax.lax.dot_general(
        agg_ref[...], lw_ref[...], (((1,), (0,)), ((), ())),
        precision=HI, preferred_element_type=jnp.float32) + lb_ref[...]


def _att_fast(xw, xj_flat, nbr, num_nodes, lwp, lbp, heads, cp, c_true):
    Dp = _round128(heads * cp)
    outp = lwp.shape[1]
    xj = xj_flat.reshape(KSLOT, NP_PAD, Dp)
    grid = NP_PAD // BLK
    body = functools.partial(_att_body, heads=heads, cp=cp, c_true=c_true)
    return pl.pallas_call(
        body,
        grid=(grid,),
        in_specs=[
            pl.BlockSpec((KSLOT, BLK, Dp), lambda i: (0, i, 0)),
            pl.BlockSpec((BLK, Dp), lambda i: (i, 0)),
            pl.BlockSpec((BLK, 32), lambda i: (i, 0)),
            pl.BlockSpec(memory_space=pltpu.SMEM),
            pl.BlockSpec((heads * cp, outp), lambda i: (0, 0)),
            pl.BlockSpec((1, outp), lambda i: (0, 0)),
        ],
        out_specs=pl.BlockSpec((BLK, outp), lambda i: (i, 0)),
        out_shape=jax.ShapeDtypeStruct((NP_PAD, outp), jnp.float32),
        scratch_shapes=[pltpu.VMEM((BLK, heads * cp), jnp.float32)],
    )(xj, xw, nbr, num_nodes.reshape(1), lwp, lbp.reshape(1, outp))


def _att_slow(x, nbr, num_nodes, W, lw, lb, heads, out_c):
    N = N_REAL
    kept = nbr[:N, :MAX_K] != N
    q = jnp.broadcast_to(jnp.arange(N, dtype=jnp.int32)[:, None], (N, MAX_K))
    ei0 = nbr[:N, :MAX_K].reshape(-1)
    ei1 = jnp.where(kept, q, 0).reshape(-1)
    loop = jnp.arange(N, dtype=jnp.int32)
    lk = loop < num_nodes
    ei = jnp.concatenate([ei0, jnp.where(lk, loop, N)])
    ej = jnp.concatenate([ei1, jnp.where(lk, loop, 0)])
    xw = x[:N] @ W
    xi = xw[jnp.minimum(ei, N - 1)].reshape(-1, heads, out_c)
    xj = xw[ej].reshape(-1, heads, out_c)
    scores = jnp.einsum('ehc,egc->eh', xi, xj) / jnp.sqrt(
        jnp.asarray(out_c, jnp.float32))
    m = jax.ops.segment_max(scores, ei, num_segments=N + 1)
    m = jnp.where(jnp.isfinite(m), m, 0.0)
    e = jnp.exp(scores - m[ei])
    s = jax.ops.segment_sum(e, ei, num_segments=N + 1)
    alpha = e / (s[ei] + 1e-16)
    msg = (xj * alpha[:, :, None]).reshape(-1, heads * out_c)
    agg = jax.ops.segment_max(msg, ei, num_segments=N + 1)
    agg = jnp.where(jnp.isfinite(agg), agg, 0.0)
    return agg[:N] @ lw + lb


def _fc_pool_call(x1p, x2p, batp, fc1a, fc1b, fc1bias, bng, bnb):
    grid = NP_PAD // 256

    def body(x1_ref, x2_ref, bat_ref, w1_ref, w2_ref, b_ref, g_ref, bb_ref, o_ref):
        i0 = pl.program_id(0)
        h = (jax.lax.dot_general(x1_ref[...], w1_ref[...], (((1,), (0,)), ((), ())),
                                 precision=HI, preferred_element_type=jnp.float32)
             + jax.lax.dot_general(x2_ref[...], w2_ref[...], (((1,), (0,)), ((), ())),
                                   precision=HI, preferred_element_type=jnp.float32)
             + b_ref[...])
        h = jnp.maximum(h, 0.0)
        h = g_ref[...] * h * (1.0 / jnp.sqrt(1.0 + BN_EPS)) + bb_ref[...]

        @pl.when(i0 == 0)
        def _():
            o_ref[...] = jnp.full_like(o_ref[...], -jnp.inf)

        bat = bat_ref[...]
        cur = o_ref[...]
        parts = []
        for g in range(8):
            sel = bat == g
            parts.append(jnp.max(jnp.where(sel, h, -jnp.inf), axis=0,
                                 keepdims=True))
        o_ref[...] = jnp.maximum(cur, jnp.concatenate(parts, axis=0))

    return pl.pallas_call(
        body,
        grid=(grid,),
        in_specs=[
            pl.BlockSpec((256, x1p.shape[1]), lambda i: (i, 0)),
            pl.BlockSpec((256, x2p.shape[1]), lambda i: (i, 0)),
            pl.BlockSpec((256, 1), lambda i: (i, 0)),
            pl.BlockSpec((x1p.shape[1], 128), lambda i: (0, 0)),
            pl.BlockSpec((x2p.shape[1], 128), lambda i: (0, 0)),
            pl.BlockSpec((1, 128), lambda i: (0, 0)),
            pl.BlockSpec((1, 128), lambda i: (0, 0)),
            pl.BlockSpec((1, 128), lambda i: (0, 0)),
        ],
        out_specs=pl.BlockSpec((8, 128), lambda i: (0, 0)),
        out_shape=jax.ShapeDtypeStruct((8, 128), jnp.float32),
    )(x1p, x2p, batp.reshape(NP_PAD, 1), fc1a, fc1b, fc1bias.reshape(1, 128),
      bng.reshape(1, 128), bnb.reshape(1, 128))


def _bn_eval(x, g, b):
    return g * x / jnp.sqrt(1.0 + BN_EPS) + b


def _head_pallas(g, p):
    def body(g_ref, m1w, m1b, mg1, mb1, m2w, m2b, mg2, mb2, ow, ob, out_ref):
        gg = jnp.where(jnp.isfinite(g_ref[...]), g_ref[...], 0.0)
        h = jnp.maximum(jnp.dot(gg, m1w[...],
                                precision=HI) + m1b[...], 0.0)
        h = _bn_eval(h, mg1[...], mb1[...])
        h = jnp.maximum(jnp.dot(h, m2w[...], precision=HI) + m2b[...], 0.0)
        h = _bn_eval(h, mg2[...], mb2[...])
        logits = jnp.dot(h, ow[...], precision=HI) + ob[...]
        mx = jnp.max(logits, axis=1, keepdims=True)
        lse = jnp.log(jnp.sum(jnp.exp(logits - mx), axis=1, keepdims=True)) + mx
        out_ref[...] = logits - lse

    args = (g, p['m1_w'], p['m1_b'].reshape(1, -1), p['mbn1_g'].reshape(1, -1),
            p['mbn1_b'].reshape(1, -1), p['m2_w'], p['m2_b'].reshape(1, -1),
            p['mbn2_g'].reshape(1, -1), p['mbn2_b'].reshape(1, -1),
            p['out_w'], p['out_b'].reshape(1, -1))
    return pl.pallas_call(
        body,
        out_shape=jax.ShapeDtypeStruct((g.shape[0], NUM_CLASSES), jnp.float32),
    )(*args)


def _round128(n):
    return ((n + 127) // 128) * 128


def _pad_w_per_head(W, heads, c, cp):
    d = W.shape[0]
    Wr = W.reshape(d, heads, c)
    Wp = jnp.pad(Wr, ((0, 0), (0, 0), (0, cp - c))).reshape(d, heads * cp)
    return jnp.pad(Wp, ((0, 0), (0, _round128(heads * cp) - heads * cp)))


def _pad_lw_rows(lw, heads, c, cp, outp):
    hc, out = lw.shape
    lwr = lw.reshape(heads, c, out)
    lwr = jnp.pad(lwr, ((0, 0), (0, cp - c), (0, 0))).reshape(heads * cp, out)
    return jnp.pad(lwr, ((0, 0), (0, outp - out)))


def _gat_layer(x, xp, batp, W, lw, lb, heads, c, cp, outp):
    nbr, rowmx, cnt = _radius_topk_call(xp, batp)
    num_nodes = jnp.max(rowmx) + 1
    overflow = jnp.max(cnt[:N_REAL]) > MAX_K

    Wp = _pad_w_per_head(W, heads, c, cp)
    lwp = _pad_lw_rows(lw, heads, c, cp, outp)
    lbp = jnp.pad(lb, (0, outp - lb.shape[0]))

    def fast(_):
        xw = _xw_call(xp, Wp)
        rid = jnp.arange(NP_PAD, dtype=jnp.int32)
        gidx = jnp.concatenate(
            [jnp.where(nbr[:, :MAX_K] == N_REAL, 0, nbr[:, :MAX_K]).T,
             rid[None, :]], axis=0).reshape(-1)
        xj_flat = _sc_gather(xw, gidx)
        return _att_fast(xw, xj_flat, nbr, num_nodes, lwp, lbp, heads, cp, c)

    def slow(_):
        out = _att_slow(x, nbr, num_nodes, W, lw, lb, heads, c)
        outp_arr = jnp.zeros((NP_PAD, outp), jnp.float32)
        return outp_arr.at[:N_REAL, :lb.shape[0]].set(out)

    return lax.cond(overflow, slow, fast, operand=None)


def kernel(pos, batch, params):
    xp1, batp = _pad_inputs(pos, batch)
    CP1 = 32
    x1p = _gat_layer(pos, xp1, batp, params['W1'], params['lin1_w'],
                     params['lin1_b'], H1, C1, CP1, 32)
    x1 = x1p[:N_REAL, :C1]
    xp2 = jnp.zeros((NP_PAD, C1), jnp.float32).at[:N_REAL].set(x1)
    x2p = _gat_layer(x1, xp2, batp, params['W2'], params['lin2_w'],
                     params['lin2_b'], H2, C2, C2, 64)

    fc1a = jnp.pad(params['fc1_w'][:C1], ((0, 32 - C1), (0, 0)))
    fc1b = params['fc1_w'][C1:]
    g = _fc_pool_call(x1p, x2p, batp, fc1a, fc1b, params['fc1_b'],
                      params['bn1_g'], params['bn1_b'])
    return _head_pallas(g, params)
